# Initial kernel scaffold; baseline (speedup 1.0000x reference)
import jax
import jax.numpy as jnp
from jax.experimental import pallas as pl


def kernel(x, id_weight, level_weight):
    raise NotImplementedError("write your pallas kernel here")


# speedup vs baseline: 7.5531x; 7.5531x over previous
import jax
import jax.numpy as jnp
from jax.experimental import pallas as pl
from jax.experimental.pallas import tpu as pltpu

_D = 10000
_L = 100
_S = 784
_B = 8
_SBLK = 392
_NSTEPS = _S // _SBLK


def _body(x_ref, id_ref, lvl_ref, out_ref, acc_ref):
    i = pl.program_id(0)
    xv = x_ref[0]
    idx_s = jnp.clip(jnp.round(jnp.clip(xv, 0.0, 1.0) * (_L - 1)), 0, _L - 1).astype(jnp.int32)
    id_bf = id_ref[...].astype(jnp.bfloat16)
    lvl = lvl_ref[...]
    liota3 = jax.lax.broadcasted_iota(jnp.int32, (_B, _L, _SBLK), 1)
    m = (idx_s[:, None, :] == liota3).astype(jnp.bfloat16).reshape(_B * _L, _SBLK)
    s = jax.lax.dot(m, id_bf, preferred_element_type=jnp.float32)
    contrib = jnp.stack(
        [jnp.sum(lvl * s[b * _L:(b + 1) * _L, :], axis=0) for b in range(_B)]
    )

    @pl.when(i == 0)
    def _():
        acc_ref[...] = contrib

    @pl.when(i > 0)
    def _():
        acc_ref[...] += contrib

    @pl.when(i == _NSTEPS - 1)
    def _():
        out_ref[...] = jnp.where(acc_ref[...] > 0, 1.0, -1.0)


def kernel(x, id_weight, level_weight):
    x3 = jnp.transpose(x.reshape(_B, _NSTEPS, _SBLK), (1, 0, 2))
    return pl.pallas_call(
        _body,
        grid=(_NSTEPS,),
        in_specs=[
            pl.BlockSpec((1, _B, _SBLK), lambda i: (i, 0, 0)),
            pl.BlockSpec((_SBLK, _D), lambda i: (i, 0)),
            pl.BlockSpec((_L, _D), lambda i: (0, 0)),
        ],
        out_specs=pl.BlockSpec((_B, _D), lambda i: (0, 0)),
        out_shape=jax.ShapeDtypeStruct((_B, _D), jnp.float32),
        scratch_shapes=[pltpu.VMEM((_B, _D), jnp.float32)],
    )(x3, id_weight, level_weight)
